# Initial kernel scaffold; baseline (speedup 1.0000x reference)
import jax
import jax.numpy as jnp
from jax.experimental import pallas as pl


def kernel(x, edge_index, batch_lengths, W_emb, b_emb, W_ih, W_hh, b_ih, b_hh, gcn_W0, gcn_b0, gcn_W1, gcn_b1, cls_W, cls_b):
    raise NotImplementedError("write your pallas kernel here")


# speedup vs baseline: 4.0814x; 4.0814x over previous
import functools

import jax
import jax.numpy as jnp
from jax import lax
from jax.experimental import pallas as pl
from jax.experimental.pallas import tpu as pltpu
from jax.experimental.pallas import tpu_sc as plsc

N = 50000
E = 800000
L = 8
D_IN = 64
H = 64
OUT = 64

NC = 2
NS = 16
ROWS_PT = N // NS
EDGES_PT = E // NS
CH = 128
FULL_CHUNKS = EDGES_PT // CH
TAIL = EDGES_PT - FULL_CHUNKS * CH

BN = 2000
GRID_N = N // BN


def _lstm_body(x_ref, len_ref, we_ref, be_ref, wih_ref, whh_ref, bg_ref,
               out_ref):
    lens = len_ref[...]
    h = jnp.zeros((BN, H), jnp.float32)
    c = jnp.zeros((BN, H), jnp.float32)
    we = we_ref[...]
    wih = wih_ref[...]
    whh = whh_ref[...]
    bg = bg_ref[...]
    be = be_ref[...]
    for t in range(L):
        xt = x_ref[:, t, :]
        xe = jnp.dot(xt, we, preferred_element_type=jnp.float32) + be
        gates = (jnp.dot(xe, wih, preferred_element_type=jnp.float32)
                 + jnp.dot(h, whh, preferred_element_type=jnp.float32) + bg)
        i_g = jax.nn.sigmoid(gates[:, 0:H])
        f_g = jax.nn.sigmoid(gates[:, H:2 * H])
        g_g = jnp.tanh(gates[:, 2 * H:3 * H])
        o_g = jax.nn.sigmoid(gates[:, 3 * H:4 * H])
        c_new = f_g * c + i_g * g_g
        h_new = o_g * jnp.tanh(c_new)
        m = lens > t
        h = jnp.where(m, h_new, h)
        c = jnp.where(m, c_new, c)
    out_ref[...] = h


def _lstm_call(x, lens, we, be, wih, whh, bg):
    return pl.pallas_call(
        _lstm_body,
        grid=(GRID_N,),
        in_specs=[
            pl.BlockSpec((BN, L, D_IN), lambda i: (i, 0, 0)),
            pl.BlockSpec((BN, 1), lambda i: (i, 0)),
            pl.BlockSpec((D_IN, H), lambda i: (0, 0)),
            pl.BlockSpec((1, H), lambda i: (0, 0)),
            pl.BlockSpec((H, 4 * H), lambda i: (0, 0)),
            pl.BlockSpec((H, 4 * H), lambda i: (0, 0)),
            pl.BlockSpec((1, 4 * H), lambda i: (0, 0)),
        ],
        out_specs=pl.BlockSpec((BN, H), lambda i: (i, 0)),
        out_shape=jax.ShapeDtypeStruct((N, H), jnp.float32),
    )(x, lens, we, be, wih, whh, bg)


def _deg_body(idx2_hbm, ones_hbm, zeros_hbm, out_hbm, acc, idx_v, idx_t,
              ones_v):
    cid = lax.axis_index("c")
    sid = lax.axis_index("s")
    pltpu.sync_copy(zeros_hbm, acc.at[pl.ds(sid * ROWS_PT, ROWS_PT)])
    pltpu.sync_copy(ones_hbm, ones_v)
    plsc.subcore_barrier()

    base0 = cid * E + sid * EDGES_PT

    def body(j, carry):
        base = base0 + j * CH
        pltpu.sync_copy(idx2_hbm.at[pl.ds(base, CH)], idx_v)
        pltpu.sync_copy(ones_v, acc.at[idx_v], add=True)
        return carry

    lax.fori_loop(0, FULL_CHUNKS, body, 0)
    tbase = base0 + FULL_CHUNKS * CH
    pltpu.sync_copy(idx2_hbm.at[pl.ds(tbase, TAIL)], idx_t)
    pltpu.sync_copy(ones_v.at[pl.ds(0, TAIL)], acc.at[idx_t], add=True)
    plsc.subcore_barrier()

    @pl.when(sid == 0)
    def _():
        pltpu.sync_copy(acc, out_hbm.at[cid])


def _deg_call(idx2, ones16, zeros16):
    mesh = plsc.VectorSubcoreMesh(core_axis_name="c", subcore_axis_name="s",
                                  num_cores=NC, num_subcores=NS)
    f = pl.kernel(
        _deg_body,
        out_type=jax.ShapeDtypeStruct((2, N, 16), jnp.float32),
        mesh=mesh,
        scratch_types=[
            pltpu.VMEM_SHARED((N, 16), jnp.float32),
            pltpu.VMEM((CH,), jnp.int32),
            pltpu.VMEM((TAIL,), jnp.int32),
            pltpu.VMEM((CH, 16), jnp.float32),
        ],
        compiler_params=pltpu.CompilerParams(use_tc_tiling_on_sc=False),
    )
    return f(idx2, ones16, zeros16)


def _agg_body(feat_hbm, src2_hbm, dst_hbm, zeros_hbm, out_hbm, acc,
              sidx_v, didx_v, sidx_t, didx_t, rows_v, rows_t, sem):
    cid = lax.axis_index("c")
    sid = lax.axis_index("s")
    pltpu.sync_copy(zeros_hbm, acc.at[pl.ds(sid * ROWS_PT, ROWS_PT)])
    plsc.subcore_barrier()

    sbase0 = cid * E + sid * EDGES_PT
    dbase0 = sid * EDGES_PT

    def body(j, carry):
        off = j * CH
        pltpu.sync_copy(src2_hbm.at[pl.ds(sbase0 + off, CH)], sidx_v)
        pltpu.sync_copy(dst_hbm.at[pl.ds(dbase0 + off, CH)], didx_v)
        pltpu.async_copy(feat_hbm.at[sidx_v], rows_v, sem).wait()
        pltpu.sync_copy(rows_v, acc.at[didx_v], add=True)
        return carry

    lax.fori_loop(0, FULL_CHUNKS, body, 0)
    toff = FULL_CHUNKS * CH
    pltpu.sync_copy(src2_hbm.at[pl.ds(sbase0 + toff, TAIL)], sidx_t)
    pltpu.sync_copy(dst_hbm.at[pl.ds(dbase0 + toff, TAIL)], didx_t)
    pltpu.async_copy(feat_hbm.at[sidx_t], rows_t, sem).wait()
    pltpu.sync_copy(rows_t, acc.at[didx_t], add=True)
    plsc.subcore_barrier()

    @pl.when(sid == 0)
    def _():
        pltpu.sync_copy(acc, out_hbm.at[cid])


def _agg_call(feat_flat, src2, dst, zeros32):
    mesh = plsc.VectorSubcoreMesh(core_axis_name="c", subcore_axis_name="s",
                                  num_cores=NC, num_subcores=NS)
    f = pl.kernel(
        _agg_body,
        out_type=jax.ShapeDtypeStruct((2, N, 32), jnp.float32),
        mesh=mesh,
        scratch_types=[
            pltpu.VMEM_SHARED((N, 32), jnp.float32),
            pltpu.VMEM((CH,), jnp.int32),
            pltpu.VMEM((CH,), jnp.int32),
            pltpu.VMEM((TAIL,), jnp.int32),
            pltpu.VMEM((TAIL,), jnp.int32),
            pltpu.VMEM((CH, 32), jnp.float32),
            pltpu.VMEM((TAIL, 32), jnp.float32),
            pltpu.SemaphoreType.DMA,
        ],
        compiler_params=pltpu.CompilerParams(use_tc_tiling_on_sc=False),
    )
    return f(feat_flat, src2, dst, zeros32)


def _feat_body(h_ref, dout_ref, w_ref, out_ref):
    s = lax.rsqrt(jnp.maximum(dout_ref[...][:, 0:1], 1.0))
    h = h_ref[...]
    out_ref[0] = jnp.dot(h, w_ref[0], preferred_element_type=jnp.float32) * s
    out_ref[1] = jnp.dot(h, w_ref[1], preferred_element_type=jnp.float32) * s


def _feat_call(h, dout, w_split):
    return pl.pallas_call(
        _feat_body,
        grid=(GRID_N,),
        in_specs=[
            pl.BlockSpec((BN, H), lambda i: (i, 0)),
            pl.BlockSpec((BN, 16), lambda i: (i, 0)),
            pl.BlockSpec((2, H, 32), lambda i: (0, 0, 0)),
        ],
        out_specs=pl.BlockSpec((2, BN, 32), lambda i: (0, i, 0)),
        out_shape=jax.ShapeDtypeStruct((2, N, 32), jnp.float32),
    )(h, dout, w_split)


def _mid_body(agg_ref, din_ref, dout_ref, w_ref, b_ref, out_ref):
    s_in = lax.rsqrt(jnp.maximum(din_ref[...][:, 0:1], 1.0))
    s_out = lax.rsqrt(jnp.maximum(dout_ref[...][:, 0:1], 1.0))
    full = jnp.concatenate([agg_ref[0], agg_ref[1]], axis=1)
    h1 = jax.nn.relu(full * s_in + b_ref[...])
    out_ref[0] = jnp.dot(h1, w_ref[0],
                         preferred_element_type=jnp.float32) * s_out
    out_ref[1] = jnp.dot(h1, w_ref[1],
                         preferred_element_type=jnp.float32) * s_out


def _mid_call(agg, din, dout, w_split, b):
    return pl.pallas_call(
        _mid_body,
        grid=(GRID_N,),
        in_specs=[
            pl.BlockSpec((2, BN, 32), lambda i: (0, i, 0)),
            pl.BlockSpec((BN, 16), lambda i: (i, 0)),
            pl.BlockSpec((BN, 16), lambda i: (i, 0)),
            pl.BlockSpec((2, OUT, 32), lambda i: (0, 0, 0)),
            pl.BlockSpec((1, OUT), lambda i: (0, 0)),
        ],
        out_specs=pl.BlockSpec((2, BN, 32), lambda i: (0, i, 0)),
        out_shape=jax.ShapeDtypeStruct((2, N, 32), jnp.float32),
    )(agg, din, dout, w_split, b)


def _pool_body(agg_ref, din_ref, b_ref, wc_ref, bc_ref, out_ref, acc):
    i = pl.program_id(0)

    @pl.when(i == 0)
    def _():
        acc[...] = jnp.zeros((1, OUT), jnp.float32)

    s_in = lax.rsqrt(jnp.maximum(din_ref[...][:, 0:1], 1.0))
    full = jnp.concatenate([agg_ref[0], agg_ref[1]], axis=1)
    h2 = jax.nn.relu(full * s_in + b_ref[...])
    acc[...] += jnp.sum(h2, axis=0, keepdims=True)

    @pl.when(i == GRID_N - 1)
    def _():
        hg = acc[...] * (1.0 / N)
        out_ref[...] = jnp.dot(hg, wc_ref[...],
                               preferred_element_type=jnp.float32) + bc_ref[...]


def _pool_call(agg, din, b, wc, bc):
    return pl.pallas_call(
        _pool_body,
        grid=(GRID_N,),
        in_specs=[
            pl.BlockSpec((2, BN, 32), lambda i: (0, i, 0)),
            pl.BlockSpec((BN, 16), lambda i: (i, 0)),
            pl.BlockSpec((1, OUT), lambda i: (0, 0)),
            pl.BlockSpec((OUT, 2), lambda i: (0, 0)),
            pl.BlockSpec((1, 2), lambda i: (0, 0)),
        ],
        out_specs=pl.BlockSpec((1, 2), lambda i: (0, 0)),
        out_shape=jax.ShapeDtypeStruct((1, 2), jnp.float32),
        scratch_shapes=[pltpu.VMEM((1, OUT), jnp.float32)],
    )(agg, din, b, wc, bc)


@jax.jit
def _run(x, edge_index, batch_lengths, W_emb, b_emb, W_ih, W_hh, b_ih, b_hh,
         gcn_W0, gcn_b0, gcn_W1, gcn_b1, cls_W, cls_b):
    src = edge_index[0]
    dst = edge_index[1]
    src2 = jnp.concatenate([src, src + N])
    degidx = jnp.concatenate([src, dst])
    lens = batch_lengths.reshape(N, 1)

    we = W_emb.T
    be = b_emb.reshape(1, H)
    wih = W_ih.T
    whh = W_hh.T
    bg = (b_ih + b_hh).reshape(1, 4 * H)
    w0s = jnp.stack([gcn_W0[:, :32], gcn_W0[:, 32:]])
    w1s = jnp.stack([gcn_W1[:, :32], gcn_W1[:, 32:]])
    b0 = gcn_b0.reshape(1, OUT)
    b1 = gcn_b1.reshape(1, OUT)
    wc = cls_W.T
    bc = cls_b.reshape(1, 2)

    ones16 = jnp.ones((CH, 16), jnp.float32)
    zeros16 = jnp.zeros((ROWS_PT, 16), jnp.float32)
    zeros32 = jnp.zeros((ROWS_PT, 32), jnp.float32)

    hn = _lstm_call(x, lens, we, be, wih, whh, bg)
    degp = _deg_call(degidx, ones16, zeros16)
    dout = degp[0]
    din = degp[1]

    feat = _feat_call(hn, dout, w0s)
    agg = _agg_call(feat.reshape(2 * N, 32), src2, dst, zeros32)
    feat2 = _mid_call(agg, din, dout, w1s, b0)
    agg2 = _agg_call(feat2.reshape(2 * N, 32), src2, dst, zeros32)
    return _pool_call(agg2, din, b1, wc, bc)


def kernel(x, edge_index, batch_lengths, W_emb, b_emb, W_ih, W_hh, b_ih, b_hh,
           gcn_W0, gcn_b0, gcn_W1, gcn_b1, cls_W, cls_b):
    return _run(x, edge_index, batch_lengths, W_emb, b_emb, W_ih, W_hh,
                b_ih, b_hh, gcn_W0, gcn_b0, gcn_W1, gcn_b1, cls_W, cls_b)


# speedup vs baseline: 5.8185x; 1.4256x over previous
import functools

import jax
import jax.numpy as jnp
from jax import lax
from jax.experimental import pallas as pl
from jax.experimental.pallas import tpu as pltpu
from jax.experimental.pallas import tpu_sc as plsc

N = 50000
E = 800000
L = 8
D_IN = 64
H = 64
OUT = 64

NC = 2
NS = 16
ROWS_PT = N // NS
CH = 128
R2D = E // CH
RPT_ROWS = R2D // NS
SB = 10
NSB = RPT_ROWS // SB
SB_A = 2
NSB_A = RPT_ROWS // SB_A
EXTRA = R2D - NS * RPT_ROWS

BN = 2000
GRID_N = N // BN


def _lstm_body(x_ref, len_ref, we_ref, be_ref, wih_ref, whh_ref, bg_ref,
               out_ref):
    lens = len_ref[...]
    h = jnp.zeros((BN, H), jnp.float32)
    c = jnp.zeros((BN, H), jnp.float32)
    we = we_ref[...]
    wih = wih_ref[...]
    whh = whh_ref[...]
    bg = bg_ref[...]
    be = be_ref[...]
    for t in range(L):
        xt = x_ref[:, t, :]
        xe = jnp.dot(xt, we, preferred_element_type=jnp.float32) + be
        gates = (jnp.dot(xe, wih, preferred_element_type=jnp.float32)
                 + jnp.dot(h, whh, preferred_element_type=jnp.float32) + bg)
        i_g = jax.nn.sigmoid(gates[:, 0:H])
        f_g = jax.nn.sigmoid(gates[:, H:2 * H])
        g_g = jnp.tanh(gates[:, 2 * H:3 * H])
        o_g = jax.nn.sigmoid(gates[:, 3 * H:4 * H])
        c_new = f_g * c + i_g * g_g
        h_new = o_g * jnp.tanh(c_new)
        m = lens > t
        h = jnp.where(m, h_new, h)
        c = jnp.where(m, c_new, c)
    out_ref[...] = h


def _lstm_call(x, lens, we, be, wih, whh, bg):
    return pl.pallas_call(
        _lstm_body,
        grid=(GRID_N,),
        in_specs=[
            pl.BlockSpec((BN, L, D_IN), lambda i: (i, 0, 0)),
            pl.BlockSpec((BN, 1), lambda i: (i, 0)),
            pl.BlockSpec((D_IN, H), lambda i: (0, 0)),
            pl.BlockSpec((1, H), lambda i: (0, 0)),
            pl.BlockSpec((H, 4 * H), lambda i: (0, 0)),
            pl.BlockSpec((H, 4 * H), lambda i: (0, 0)),
            pl.BlockSpec((1, 4 * H), lambda i: (0, 0)),
        ],
        out_specs=pl.BlockSpec((BN, H), lambda i: (i, 0)),
        out_shape=jax.ShapeDtypeStruct((N, H), jnp.float32),
    )(x, lens, we, be, wih, whh, bg)


def _deg_body(idx2_hbm, ones_hbm, zeros_hbm, out_hbm, acc, didx, ones_v,
              ssem):
    cid = lax.axis_index("c")
    sid = lax.axis_index("s")
    pltpu.sync_copy(zeros_hbm, acc.at[pl.ds(sid * ROWS_PT, ROWS_PT)])
    pltpu.sync_copy(ones_hbm, ones_v)
    plsc.subcore_barrier()

    base = cid * R2D + sid * RPT_ROWS

    def drain(p):
        for r in range(SB):
            pltpu.make_async_copy(ones_v, acc.at[didx.at[p, r]],
                                  ssem.at[p]).wait()

    def sb_block(row0, p, maybe_drain):
        if maybe_drain is True:
            drain(p)
        elif maybe_drain is not None:
            @pl.when(maybe_drain)
            def _():
                drain(p)
        pltpu.sync_copy(idx2_hbm.at[pl.ds(row0, SB)], didx.at[p])
        for r in range(SB):
            pltpu.async_copy(ones_v, acc.at[didx.at[p, r]], ssem.at[p],
                             add=True)

    def body(m, carry):
        row0 = base + (2 * m) * SB
        sb_block(row0, 0, m >= 1)
        sb_block(row0 + SB, 1, m >= 1)
        return carry

    lax.fori_loop(0, (NSB - 1) // 2, body, 0)
    sb_block(base + (NSB - 1) * SB, 0, True)
    drain(0)
    drain(1)
    @pl.when(sid < EXTRA)
    def _():
        er = cid * R2D + NS * RPT_ROWS + sid
        pltpu.sync_copy(idx2_hbm.at[pl.ds(er, 1)], didx.at[0, pl.ds(0, 1)])
        pltpu.sync_copy(ones_v, acc.at[didx.at[0, 0]], add=True)

    plsc.subcore_barrier()

    @pl.when(sid == 0)
    def _():
        pltpu.sync_copy(acc, out_hbm.at[cid])


def _deg_call(idx2, ones16, zeros16):
    mesh = plsc.VectorSubcoreMesh(core_axis_name="c", subcore_axis_name="s",
                                  num_cores=NC, num_subcores=NS)
    f = pl.kernel(
        _deg_body,
        out_type=jax.ShapeDtypeStruct((2, N, 16), jnp.float32),
        mesh=mesh,
        scratch_types=[
            pltpu.VMEM_SHARED((N, 16), jnp.float32),
            pltpu.VMEM((2, SB, CH), jnp.int32),
            pltpu.VMEM((CH, 16), jnp.float32),
            pltpu.SemaphoreType.DMA((2,)),
        ],
        compiler_params=pltpu.CompilerParams(use_tc_tiling_on_sc=False),
    )
    return f(idx2, ones16, zeros16)


def _agg_body(feat_hbm, src2_hbm, dst_hbm, zeros_hbm, out_hbm, acc,
              sidx, didx, rows, gsem, ssem):
    cid = lax.axis_index("c")
    sid = lax.axis_index("s")
    pltpu.sync_copy(zeros_hbm, acc.at[pl.ds(sid * ROWS_PT, ROWS_PT)])
    plsc.subcore_barrier()

    sbase = cid * R2D + sid * RPT_ROWS
    dbase = sid * RPT_ROWS

    def drain(p):
        for r in range(SB_A):
            pltpu.make_async_copy(rows.at[p, r], acc.at[didx.at[p, r]],
                                  ssem.at[p]).wait()

    def sb_block(soff, p, maybe_drain):
        if maybe_drain is True:
            drain(p)
        elif maybe_drain is not None:
            @pl.when(maybe_drain)
            def _():
                drain(p)
        pltpu.sync_copy(src2_hbm.at[pl.ds(sbase + soff, SB_A)], sidx.at[p])
        pltpu.sync_copy(dst_hbm.at[pl.ds(dbase + soff, SB_A)], didx.at[p])
        descs = [pltpu.async_copy(feat_hbm.at[sidx.at[p, r]], rows.at[p, r],
                                  gsem) for r in range(SB_A)]
        for d in descs:
            d.wait()
        for r in range(SB_A):
            pltpu.async_copy(rows.at[p, r], acc.at[didx.at[p, r]],
                             ssem.at[p], add=True)

    def body(m, carry):
        soff = (2 * m) * SB_A
        sb_block(soff, 0, m >= 1)
        sb_block(soff + SB_A, 1, m >= 1)
        return carry

    lax.fori_loop(0, (NSB_A - 1) // 2, body, 0)
    sb_block((NSB_A - 1) * SB_A, 0, True)
    drain(0)
    drain(1)
    @pl.when(sid < EXTRA)
    def _():
        ser = cid * R2D + NS * RPT_ROWS + sid
        der = NS * RPT_ROWS + sid
        pltpu.sync_copy(src2_hbm.at[pl.ds(ser, 1)], sidx.at[0, pl.ds(0, 1)])
        pltpu.sync_copy(dst_hbm.at[pl.ds(der, 1)], didx.at[0, pl.ds(0, 1)])
        pltpu.async_copy(feat_hbm.at[sidx.at[0, 0]], rows.at[0, 0],
                         gsem).wait()
        pltpu.sync_copy(rows.at[0, 0], acc.at[didx.at[0, 0]], add=True)

    plsc.subcore_barrier()

    @pl.when(sid == 0)
    def _():
        pltpu.sync_copy(acc, out_hbm.at[cid])


def _agg_call(feat_flat, src2, dst, zeros32):
    mesh = plsc.VectorSubcoreMesh(core_axis_name="c", subcore_axis_name="s",
                                  num_cores=NC, num_subcores=NS)
    f = pl.kernel(
        _agg_body,
        out_type=jax.ShapeDtypeStruct((2, N, 32), jnp.float32),
        mesh=mesh,
        scratch_types=[
            pltpu.VMEM_SHARED((N, 32), jnp.float32),
            pltpu.VMEM((2, SB_A, CH), jnp.int32),
            pltpu.VMEM((2, SB_A, CH), jnp.int32),
            pltpu.VMEM((2, SB_A, CH, 32), jnp.float32),
            pltpu.SemaphoreType.DMA,
            pltpu.SemaphoreType.DMA((2,)),
        ],
        compiler_params=pltpu.CompilerParams(use_tc_tiling_on_sc=False),
    )
    return f(feat_flat, src2, dst, zeros32)


def _feat_body(h_ref, dout_ref, w_ref, out_ref):
    s = lax.rsqrt(jnp.maximum(dout_ref[...][:, 0:1], 1.0))
    h = h_ref[...]
    out_ref[0] = jnp.dot(h, w_ref[0], preferred_element_type=jnp.float32) * s
    out_ref[1] = jnp.dot(h, w_ref[1], preferred_element_type=jnp.float32) * s


def _feat_call(h, dout, w_split):
    return pl.pallas_call(
        _feat_body,
        grid=(GRID_N,),
        in_specs=[
            pl.BlockSpec((BN, H), lambda i: (i, 0)),
            pl.BlockSpec((BN, 16), lambda i: (i, 0)),
            pl.BlockSpec((2, H, 32), lambda i: (0, 0, 0)),
        ],
        out_specs=pl.BlockSpec((2, BN, 32), lambda i: (0, i, 0)),
        out_shape=jax.ShapeDtypeStruct((2, N, 32), jnp.float32),
    )(h, dout, w_split)


def _mid_body(agg_ref, din_ref, dout_ref, w_ref, b_ref, out_ref):
    s_in = lax.rsqrt(jnp.maximum(din_ref[...][:, 0:1], 1.0))
    s_out = lax.rsqrt(jnp.maximum(dout_ref[...][:, 0:1], 1.0))
    full = jnp.concatenate([agg_ref[0], agg_ref[1]], axis=1)
    h1 = jax.nn.relu(full * s_in + b_ref[...])
    out_ref[0] = jnp.dot(h1, w_ref[0],
                         preferred_element_type=jnp.float32) * s_out
    out_ref[1] = jnp.dot(h1, w_ref[1],
                         preferred_element_type=jnp.float32) * s_out


def _mid_call(agg, din, dout, w_split, b):
    return pl.pallas_call(
        _mid_body,
        grid=(GRID_N,),
        in_specs=[
            pl.BlockSpec((2, BN, 32), lambda i: (0, i, 0)),
            pl.BlockSpec((BN, 16), lambda i: (i, 0)),
            pl.BlockSpec((BN, 16), lambda i: (i, 0)),
            pl.BlockSpec((2, OUT, 32), lambda i: (0, 0, 0)),
            pl.BlockSpec((1, OUT), lambda i: (0, 0)),
        ],
        out_specs=pl.BlockSpec((2, BN, 32), lambda i: (0, i, 0)),
        out_shape=jax.ShapeDtypeStruct((2, N, 32), jnp.float32),
    )(agg, din, dout, w_split, b)


def _pool_body(agg_ref, din_ref, b_ref, wc_ref, bc_ref, out_ref, acc):
    i = pl.program_id(0)

    @pl.when(i == 0)
    def _():
        acc[...] = jnp.zeros((1, OUT), jnp.float32)

    s_in = lax.rsqrt(jnp.maximum(din_ref[...][:, 0:1], 1.0))
    full = jnp.concatenate([agg_ref[0], agg_ref[1]], axis=1)
    h2 = jax.nn.relu(full * s_in + b_ref[...])
    acc[...] += jnp.sum(h2, axis=0, keepdims=True)

    @pl.when(i == GRID_N - 1)
    def _():
        hg = acc[...] * (1.0 / N)
        out_ref[...] = jnp.dot(hg, wc_ref[...],
                               preferred_element_type=jnp.float32) + bc_ref[...]


def _pool_call(agg, din, b, wc, bc):
    return pl.pallas_call(
        _pool_body,
        grid=(GRID_N,),
        in_specs=[
            pl.BlockSpec((2, BN, 32), lambda i: (0, i, 0)),
            pl.BlockSpec((BN, 16), lambda i: (i, 0)),
            pl.BlockSpec((1, OUT), lambda i: (0, 0)),
            pl.BlockSpec((OUT, 2), lambda i: (0, 0)),
            pl.BlockSpec((1, 2), lambda i: (0, 0)),
        ],
        out_specs=pl.BlockSpec((1, 2), lambda i: (0, 0)),
        out_shape=jax.ShapeDtypeStruct((1, 2), jnp.float32),
        scratch_shapes=[pltpu.VMEM((1, OUT), jnp.float32)],
    )(agg, din, b, wc, bc)


@jax.jit
def _run(x, edge_index, batch_lengths, W_emb, b_emb, W_ih, W_hh, b_ih, b_hh,
         gcn_W0, gcn_b0, gcn_W1, gcn_b1, cls_W, cls_b):
    src = edge_index[0]
    dst = edge_index[1]
    src2 = jnp.concatenate([src, src + N]).reshape(2 * R2D, CH)
    degidx = jnp.concatenate([src, dst]).reshape(2 * R2D, CH)
    dst2d = dst.reshape(R2D, CH)
    lens = batch_lengths.reshape(N, 1)

    we = W_emb.T
    be = b_emb.reshape(1, H)
    wih = W_ih.T
    whh = W_hh.T
    bg = (b_ih + b_hh).reshape(1, 4 * H)
    w0s = jnp.stack([gcn_W0[:, :32], gcn_W0[:, 32:]])
    w1s = jnp.stack([gcn_W1[:, :32], gcn_W1[:, 32:]])
    b0 = gcn_b0.reshape(1, OUT)
    b1 = gcn_b1.reshape(1, OUT)
    wc = cls_W.T
    bc = cls_b.reshape(1, 2)

    ones16 = jnp.ones((CH, 16), jnp.float32)
    zeros16 = jnp.zeros((ROWS_PT, 16), jnp.float32)
    zeros32 = jnp.zeros((ROWS_PT, 32), jnp.float32)

    hn = _lstm_call(x, lens, we, be, wih, whh, bg)
    degp = _deg_call(degidx, ones16, zeros16)
    dout = degp[0]
    din = degp[1]

    feat = _feat_call(hn, dout, w0s)
    agg = _agg_call(feat.reshape(2 * N, 32), src2, dst2d, zeros32)
    feat2 = _mid_call(agg, din, dout, w1s, b0)
    agg2 = _agg_call(feat2.reshape(2 * N, 32), src2, dst2d, zeros32)
    return _pool_call(agg2, din, b1, wc, bc)


def kernel(x, edge_index, batch_lengths, W_emb, b_emb, W_ih, W_hh, b_ih, b_hh,
           gcn_W0, gcn_b0, gcn_W1, gcn_b1, cls_W, cls_b):
    return _run(x, edge_index, batch_lengths, W_emb, b_emb, W_ih, W_hh,
                b_ih, b_hh, gcn_W0, gcn_b0, gcn_W1, gcn_b1, cls_W, cls_b)


# speedup vs baseline: 5.8254x; 1.0012x over previous
import functools

import jax
import jax.numpy as jnp
from jax import lax
from jax.experimental import pallas as pl
from jax.experimental.pallas import tpu as pltpu
from jax.experimental.pallas import tpu_sc as plsc

N = 50000
E = 800000
L = 8
D_IN = 64
H = 64
OUT = 64

NC = 2
NS = 16
ROWS_PT = N // NS
CH = 128
R2D = E // CH
RPT_ROWS = R2D // NS
SB = 10
NSB = RPT_ROWS // SB
SB_A = 2
NSB_A = RPT_ROWS // SB_A
EXTRA = R2D - NS * RPT_ROWS

BN = 2000
GRID_N = N // BN


def _lstm_body(x_ref, len_ref, we_ref, be_ref, wih_ref, whh_ref, bg_ref,
               out_ref):
    lens = len_ref[...]
    h = jnp.zeros((BN, H), jnp.float32)
    c = jnp.zeros((BN, H), jnp.float32)
    we = we_ref[...]
    wih = wih_ref[...].astype(jnp.bfloat16)
    whh = whh_ref[...].astype(jnp.bfloat16)
    bg = bg_ref[...]
    be = be_ref[...]
    for t in range(L):
        xt = x_ref[:, t, :]
        xe = jnp.dot(xt, we, preferred_element_type=jnp.float32) + be
        gates = (jnp.dot(xe.astype(jnp.bfloat16), wih,
                         preferred_element_type=jnp.float32)
                 + jnp.dot(h.astype(jnp.bfloat16), whh,
                           preferred_element_type=jnp.float32) + bg)
        i_g = jax.nn.sigmoid(gates[:, 0:H])
        f_g = jax.nn.sigmoid(gates[:, H:2 * H])
        g_g = jnp.tanh(gates[:, 2 * H:3 * H])
        o_g = jax.nn.sigmoid(gates[:, 3 * H:4 * H])
        c_new = f_g * c + i_g * g_g
        h_new = o_g * jnp.tanh(c_new)
        m = lens > t
        h = jnp.where(m, h_new, h)
        c = jnp.where(m, c_new, c)
    out_ref[...] = h


def _lstm_call(x, lens, we, be, wih, whh, bg):
    return pl.pallas_call(
        _lstm_body,
        grid=(GRID_N,),
        in_specs=[
            pl.BlockSpec((BN, L, D_IN), lambda i: (i, 0, 0)),
            pl.BlockSpec((BN, 1), lambda i: (i, 0)),
            pl.BlockSpec((D_IN, H), lambda i: (0, 0)),
            pl.BlockSpec((1, H), lambda i: (0, 0)),
            pl.BlockSpec((H, 4 * H), lambda i: (0, 0)),
            pl.BlockSpec((H, 4 * H), lambda i: (0, 0)),
            pl.BlockSpec((1, 4 * H), lambda i: (0, 0)),
        ],
        out_specs=pl.BlockSpec((BN, H), lambda i: (i, 0)),
        out_shape=jax.ShapeDtypeStruct((N, H), jnp.float32),
    )(x, lens, we, be, wih, whh, bg)


def _deg_body(idx2_hbm, ones_hbm, zeros_hbm, out_hbm, acc, didx, ones_v,
              ssem):
    cid = lax.axis_index("c")
    sid = lax.axis_index("s")
    pltpu.sync_copy(zeros_hbm, acc.at[pl.ds(sid * ROWS_PT, ROWS_PT)])
    pltpu.sync_copy(ones_hbm, ones_v)
    plsc.subcore_barrier()

    base = cid * R2D + sid * RPT_ROWS

    def drain(p):
        for r in range(SB):
            pltpu.make_async_copy(ones_v, acc.at[didx.at[p, r]],
                                  ssem.at[p]).wait()

    def sb_block(row0, p, maybe_drain):
        if maybe_drain is True:
            drain(p)
        elif maybe_drain is not None:
            @pl.when(maybe_drain)
            def _():
                drain(p)
        pltpu.sync_copy(idx2_hbm.at[pl.ds(row0, SB)], didx.at[p])
        for r in range(SB):
            pltpu.async_copy(ones_v, acc.at[didx.at[p, r]], ssem.at[p],
                             add=True)

    def body(m, carry):
        row0 = base + (2 * m) * SB
        sb_block(row0, 0, m >= 1)
        sb_block(row0 + SB, 1, m >= 1)
        return carry

    lax.fori_loop(0, (NSB - 1) // 2, body, 0)
    sb_block(base + (NSB - 1) * SB, 0, True)
    drain(0)
    drain(1)
    @pl.when(sid < EXTRA)
    def _():
        er = cid * R2D + NS * RPT_ROWS + sid
        pltpu.sync_copy(idx2_hbm.at[pl.ds(er, 1)], didx.at[0, pl.ds(0, 1)])
        pltpu.sync_copy(ones_v, acc.at[didx.at[0, 0]], add=True)

    plsc.subcore_barrier()

    @pl.when(sid == 0)
    def _():
        pltpu.sync_copy(acc, out_hbm.at[cid])


def _deg_call(idx2, ones16, zeros16):
    mesh = plsc.VectorSubcoreMesh(core_axis_name="c", subcore_axis_name="s",
                                  num_cores=NC, num_subcores=NS)
    f = pl.kernel(
        _deg_body,
        out_type=jax.ShapeDtypeStruct((2, N, 16), jnp.float32),
        mesh=mesh,
        scratch_types=[
            pltpu.VMEM_SHARED((N, 16), jnp.float32),
            pltpu.VMEM((2, SB, CH), jnp.int32),
            pltpu.VMEM((CH, 16), jnp.float32),
            pltpu.SemaphoreType.DMA((2,)),
        ],
        compiler_params=pltpu.CompilerParams(use_tc_tiling_on_sc=False),
    )
    return f(idx2, ones16, zeros16)


def _agg_body(feat_hbm, src2_hbm, dst_hbm, zeros_hbm, out_hbm, acc,
              sidx, didx, rows, gsem, ssem):
    cid = lax.axis_index("c")
    sid = lax.axis_index("s")
    pltpu.sync_copy(zeros_hbm, acc.at[pl.ds(sid * ROWS_PT, ROWS_PT)])
    plsc.subcore_barrier()

    sbase = cid * R2D + sid * RPT_ROWS
    dbase = sid * RPT_ROWS

    def drain(p):
        for r in range(SB_A):
            pltpu.make_async_copy(rows.at[p, r], acc.at[didx.at[p, r]],
                                  ssem.at[p]).wait()

    def sb_block(soff, p, maybe_drain):
        if maybe_drain is True:
            drain(p)
        elif maybe_drain is not None:
            @pl.when(maybe_drain)
            def _():
                drain(p)
        pltpu.sync_copy(src2_hbm.at[pl.ds(sbase + soff, SB_A)], sidx.at[p])
        pltpu.sync_copy(dst_hbm.at[pl.ds(dbase + soff, SB_A)], didx.at[p])
        descs = [pltpu.async_copy(feat_hbm.at[sidx.at[p, r]], rows.at[p, r],
                                  gsem) for r in range(SB_A)]
        for d in descs:
            d.wait()
        for r in range(SB_A):
            pltpu.async_copy(rows.at[p, r], acc.at[didx.at[p, r]],
                             ssem.at[p], add=True)

    def body(m, carry):
        soff = (2 * m) * SB_A
        sb_block(soff, 0, m >= 1)
        sb_block(soff + SB_A, 1, m >= 1)
        return carry

    lax.fori_loop(0, (NSB_A - 1) // 2, body, 0)
    sb_block((NSB_A - 1) * SB_A, 0, True)
    drain(0)
    drain(1)
    @pl.when(sid < EXTRA)
    def _():
        ser = cid * R2D + NS * RPT_ROWS + sid
        der = NS * RPT_ROWS + sid
        pltpu.sync_copy(src2_hbm.at[pl.ds(ser, 1)], sidx.at[0, pl.ds(0, 1)])
        pltpu.sync_copy(dst_hbm.at[pl.ds(der, 1)], didx.at[0, pl.ds(0, 1)])
        pltpu.async_copy(feat_hbm.at[sidx.at[0, 0]], rows.at[0, 0],
                         gsem).wait()
        pltpu.sync_copy(rows.at[0, 0], acc.at[didx.at[0, 0]], add=True)

    plsc.subcore_barrier()

    @pl.when(sid == 0)
    def _():
        pltpu.sync_copy(acc, out_hbm.at[cid])


def _agg_call(feat_flat, src2, dst, zeros32):
    mesh = plsc.VectorSubcoreMesh(core_axis_name="c", subcore_axis_name="s",
                                  num_cores=NC, num_subcores=NS)
    f = pl.kernel(
        _agg_body,
        out_type=jax.ShapeDtypeStruct((2, N, 32), jnp.float32),
        mesh=mesh,
        scratch_types=[
            pltpu.VMEM_SHARED((N, 32), jnp.float32),
            pltpu.VMEM((2, SB_A, CH), jnp.int32),
            pltpu.VMEM((2, SB_A, CH), jnp.int32),
            pltpu.VMEM((2, SB_A, CH, 32), jnp.float32),
            pltpu.SemaphoreType.DMA,
            pltpu.SemaphoreType.DMA((2,)),
        ],
        compiler_params=pltpu.CompilerParams(use_tc_tiling_on_sc=False),
    )
    return f(feat_flat, src2, dst, zeros32)


def _feat_body(h_ref, dout_ref, w_ref, out_ref):
    s = lax.rsqrt(jnp.maximum(dout_ref[...][:, 0:1], 1.0))
    h = h_ref[...]
    out_ref[0] = jnp.dot(h, w_ref[0], preferred_element_type=jnp.float32) * s
    out_ref[1] = jnp.dot(h, w_ref[1], preferred_element_type=jnp.float32) * s


def _feat_call(h, dout, w_split):
    return pl.pallas_call(
        _feat_body,
        grid=(GRID_N,),
        in_specs=[
            pl.BlockSpec((BN, H), lambda i: (i, 0)),
            pl.BlockSpec((BN, 16), lambda i: (i, 0)),
            pl.BlockSpec((2, H, 32), lambda i: (0, 0, 0)),
        ],
        out_specs=pl.BlockSpec((2, BN, 32), lambda i: (0, i, 0)),
        out_shape=jax.ShapeDtypeStruct((2, N, 32), jnp.float32),
    )(h, dout, w_split)


def _mid_body(agg_ref, din_ref, dout_ref, w_ref, b_ref, out_ref):
    s_in = lax.rsqrt(jnp.maximum(din_ref[...][:, 0:1], 1.0))
    s_out = lax.rsqrt(jnp.maximum(dout_ref[...][:, 0:1], 1.0))
    full = jnp.concatenate([agg_ref[0], agg_ref[1]], axis=1)
    h1 = jax.nn.relu(full * s_in + b_ref[...])
    out_ref[0] = jnp.dot(h1, w_ref[0],
                         preferred_element_type=jnp.float32) * s_out
    out_ref[1] = jnp.dot(h1, w_ref[1],
                         preferred_element_type=jnp.float32) * s_out


def _mid_call(agg, din, dout, w_split, b):
    return pl.pallas_call(
        _mid_body,
        grid=(GRID_N,),
        in_specs=[
            pl.BlockSpec((2, BN, 32), lambda i: (0, i, 0)),
            pl.BlockSpec((BN, 16), lambda i: (i, 0)),
            pl.BlockSpec((BN, 16), lambda i: (i, 0)),
            pl.BlockSpec((2, OUT, 32), lambda i: (0, 0, 0)),
            pl.BlockSpec((1, OUT), lambda i: (0, 0)),
        ],
        out_specs=pl.BlockSpec((2, BN, 32), lambda i: (0, i, 0)),
        out_shape=jax.ShapeDtypeStruct((2, N, 32), jnp.float32),
    )(agg, din, dout, w_split, b)


def _pool_body(agg_ref, din_ref, b_ref, wc_ref, bc_ref, out_ref, acc):
    i = pl.program_id(0)

    @pl.when(i == 0)
    def _():
        acc[...] = jnp.zeros((1, OUT), jnp.float32)

    s_in = lax.rsqrt(jnp.maximum(din_ref[...][:, 0:1], 1.0))
    full = jnp.concatenate([agg_ref[0], agg_ref[1]], axis=1)
    h2 = jax.nn.relu(full * s_in + b_ref[...])
    acc[...] += jnp.sum(h2, axis=0, keepdims=True)

    @pl.when(i == GRID_N - 1)
    def _():
        hg = acc[...] * (1.0 / N)
        out_ref[...] = jnp.dot(hg, wc_ref[...],
                               preferred_element_type=jnp.float32) + bc_ref[...]


def _pool_call(agg, din, b, wc, bc):
    return pl.pallas_call(
        _pool_body,
        grid=(GRID_N,),
        in_specs=[
            pl.BlockSpec((2, BN, 32), lambda i: (0, i, 0)),
            pl.BlockSpec((BN, 16), lambda i: (i, 0)),
            pl.BlockSpec((1, OUT), lambda i: (0, 0)),
            pl.BlockSpec((OUT, 2), lambda i: (0, 0)),
            pl.BlockSpec((1, 2), lambda i: (0, 0)),
        ],
        out_specs=pl.BlockSpec((1, 2), lambda i: (0, 0)),
        out_shape=jax.ShapeDtypeStruct((1, 2), jnp.float32),
        scratch_shapes=[pltpu.VMEM((1, OUT), jnp.float32)],
    )(agg, din, b, wc, bc)


@jax.jit
def _run(x, edge_index, batch_lengths, W_emb, b_emb, W_ih, W_hh, b_ih, b_hh,
         gcn_W0, gcn_b0, gcn_W1, gcn_b1, cls_W, cls_b):
    src = edge_index[0]
    dst = edge_index[1]
    src2 = jnp.concatenate([src, src + N]).reshape(2 * R2D, CH)
    degidx = jnp.concatenate([src, dst]).reshape(2 * R2D, CH)
    dst2d = dst.reshape(R2D, CH)
    lens = batch_lengths.reshape(N, 1)

    we = W_emb.T
    be = b_emb.reshape(1, H)
    wih = W_ih.T
    whh = W_hh.T
    bg = (b_ih + b_hh).reshape(1, 4 * H)
    w0s = jnp.stack([gcn_W0[:, :32], gcn_W0[:, 32:]])
    w1s = jnp.stack([gcn_W1[:, :32], gcn_W1[:, 32:]])
    b0 = gcn_b0.reshape(1, OUT)
    b1 = gcn_b1.reshape(1, OUT)
    wc = cls_W.T
    bc = cls_b.reshape(1, 2)

    ones16 = jnp.ones((CH, 16), jnp.float32)
    zeros16 = jnp.zeros((ROWS_PT, 16), jnp.float32)
    zeros32 = jnp.zeros((ROWS_PT, 32), jnp.float32)

    hn = _lstm_call(x, lens, we, be, wih, whh, bg)
    degp = _deg_call(degidx, ones16, zeros16)
    dout = degp[0]
    din = degp[1]

    feat = _feat_call(hn, dout, w0s)
    agg = _agg_call(feat.reshape(2 * N, 32), src2, dst2d, zeros32)
    feat2 = _mid_call(agg, din, dout, w1s, b0)
    agg2 = _agg_call(feat2.reshape(2 * N, 32), src2, dst2d, zeros32)
    return _pool_call(agg2, din, b1, wc, bc)


def kernel(x, edge_index, batch_lengths, W_emb, b_emb, W_ih, W_hh, b_ih, b_hh,
           gcn_W0, gcn_b0, gcn_W1, gcn_b1, cls_W, cls_b):
    return _run(x, edge_index, batch_lengths, W_emb, b_emb, W_ih, W_hh,
                b_ih, b_hh, gcn_W0, gcn_b0, gcn_W1, gcn_b1, cls_W, cls_b)
